# baseline (speedup 1.0000x reference)
import functools

import jax
import jax.numpy as jnp
from jax.experimental import pallas as pl
from jax.experimental.pallas import tpu as pltpu

C = 1024
K = 9
D = C * K
R = 128
STEPS = C // R
FCH = 1152
SIZE = 512

_HI = jax.lax.Precision.HIGHEST


def _body(a1_ref, a2_ref, out_ref, imp1_s, cs2_s):
    i = pl.program_id(0)

    c1 = jnp.abs(a1_ref[...])
    rows = jnp.sum(c1, axis=1, keepdims=True)
    imp1_s[pl.ds(i * R, R), :] = rows

    c2 = jnp.abs(a2_ref[...])
    colpart = jnp.sum(c2, axis=0, keepdims=True)

    @pl.when(i == 0)
    def _():
        cs2_s[...] = colpart

    @pl.when(i > 0)
    def _():
        cs2_s[...] = cs2_s[...] + colpart

    @pl.when(i == STEPS - 1)
    def _():
        imp2 = jnp.zeros((C, 1), jnp.float32)
        for j in range(D // FCH):
            gidx = jax.lax.broadcasted_iota(jnp.int32, (FCH, C), 0) + j * FCH
            ch = jax.lax.broadcasted_iota(jnp.int32, (FCH, C), 1)
            fold = (gidx // K == ch).astype(jnp.float32)
            cs2c = cs2_s[:, j * FCH:(j + 1) * FCH]
            imp2 = imp2 + jax.lax.dot_general(
                fold, cs2c, (((0,), (1,)), ((), ())), precision=_HI)

        imp_col = imp1_s[...] + imp2

        eye = (jax.lax.broadcasted_iota(jnp.int32, (C, C), 0)
               == jax.lax.broadcasted_iota(jnp.int32, (C, C), 1)
               ).astype(jnp.float32)
        imp_row = jax.lax.dot_general(
            imp_col, eye, (((0,), (0,)), ((), ())), precision=_HI)

        src_i = jax.lax.broadcasted_iota(jnp.int32, (C, C), 1)
        tgt_i = jax.lax.broadcasted_iota(jnp.int32, (C, C), 0)
        cmp = jnp.where(
            (imp_row < imp_col)
            | ((imp_row == imp_col) & (src_i < tgt_i)),
            1.0, 0.0)
        rank = jnp.sum(cmp, axis=1, keepdims=True)
        maskf = jnp.where(rank < float(SIZE), 1.0, 0.0)

        lower = jnp.where(src_i < tgt_i, 1.0, 0.0)
        pos = jax.lax.dot_general(
            lower, maskf, (((1,), (0,)), ((), ())), precision=_HI)

        slot = jax.lax.broadcasted_iota(jnp.int32, (C, SIZE), 1).astype(jnp.float32)
        w = maskf * jnp.where(pos == slot, 1.0, 0.0)

        lin = (-1.0 + jax.lax.broadcasted_iota(jnp.int32, (C, 1), 0)
               .astype(jnp.float32) * (2.0 / float(C - 1)))
        out_ref[...] = jax.lax.dot_general(
            lin, w, (((0,), (0,)), ((), ())), precision=_HI)


@functools.partial(jax.jit, static_argnames=())
def _run(a1, a2):
    return pl.pallas_call(
        _body,
        grid=(STEPS,),
        in_specs=[
            pl.BlockSpec((R, D), lambda i: (i, 0)),
            pl.BlockSpec((R, D), lambda i: (i, 0)),
        ],
        out_specs=pl.BlockSpec((1, SIZE), lambda i: (0, 0)),
        out_shape=jax.ShapeDtypeStruct((1, SIZE), jnp.float32),
        scratch_shapes=[
            pltpu.VMEM((C, 1), jnp.float32),
            pltpu.VMEM((1, D), jnp.float32),
        ],
        compiler_params=pltpu.CompilerParams(
            dimension_semantics=("arbitrary",),
        ),
    )(a1, a2)


def kernel(w1, w2, size):
    a1 = w1.reshape(C, D)
    a2 = w2.reshape(C, D)
    out = _run(a1, a2)
    return out.reshape(SIZE) + size * 0


# speedup vs baseline: 1.0925x; 1.0925x over previous
import functools

import jax
import jax.numpy as jnp
from jax.experimental import pallas as pl
from jax.experimental.pallas import tpu as pltpu

C = 1024
K = 9
D = C * K
R = 128
STEPS = C // R
FCH = 1152
SIZE = 512

_HI = jax.lax.Precision.HIGHEST


def _reduce_body(a1_ref, a2_ref, imp1_ref, cs2_ref):
    i = pl.program_id(0)
    imp1_ref[...] = jnp.sum(jnp.abs(a1_ref[...]), axis=1, keepdims=True)
    colpart = jnp.sum(jnp.abs(a2_ref[...]), axis=0, keepdims=True)

    @pl.when(i == 0)
    def _():
        cs2_ref[...] = colpart

    @pl.when(i > 0)
    def _():
        cs2_ref[...] = cs2_ref[...] + colpart


def _select_body(imp1_ref, cs2_ref, out_ref):
    imp2 = jnp.zeros((C, 1), jnp.float32)
    for j in range(D // FCH):
        gidx = jax.lax.broadcasted_iota(jnp.int32, (FCH, C), 0) + j * FCH
        ch = jax.lax.broadcasted_iota(jnp.int32, (FCH, C), 1)
        fold = (gidx // K == ch).astype(jnp.float32)
        cs2c = cs2_ref[:, j * FCH:(j + 1) * FCH]
        imp2 = imp2 + jax.lax.dot_general(
            fold, cs2c, (((0,), (1,)), ((), ())), precision=_HI)

    imp_col = imp1_ref[...] + imp2

    eye = (jax.lax.broadcasted_iota(jnp.int32, (C, C), 0)
           == jax.lax.broadcasted_iota(jnp.int32, (C, C), 1)
           ).astype(jnp.float32)
    imp_row = jax.lax.dot_general(
        imp_col, eye, (((0,), (0,)), ((), ())), precision=_HI)

    src_i = jax.lax.broadcasted_iota(jnp.int32, (C, C), 1)
    tgt_i = jax.lax.broadcasted_iota(jnp.int32, (C, C), 0)
    cmp = jnp.where(
        (imp_row < imp_col) | ((imp_row == imp_col) & (src_i < tgt_i)),
        1.0, 0.0)
    rank = jnp.sum(cmp, axis=1, keepdims=True)
    maskf = jnp.where(rank < float(SIZE), 1.0, 0.0)

    lower = jnp.where(src_i < tgt_i, 1.0, 0.0)
    pos = jax.lax.dot_general(
        lower, maskf, (((1,), (0,)), ((), ())), precision=_HI)

    slot = jax.lax.broadcasted_iota(jnp.int32, (C, SIZE), 1).astype(jnp.float32)
    w = maskf * jnp.where(pos == slot, 1.0, 0.0)

    lin = (-1.0 + jax.lax.broadcasted_iota(jnp.int32, (C, 1), 0)
           .astype(jnp.float32) * (2.0 / float(C - 1)))
    out_ref[...] = jax.lax.dot_general(
        lin, w, (((0,), (0,)), ((), ())), precision=_HI)


@jax.jit
def _run(a1, a2):
    imp1, cs2 = pl.pallas_call(
        _reduce_body,
        grid=(STEPS,),
        in_specs=[
            pl.BlockSpec((R, D), lambda i: (i, 0)),
            pl.BlockSpec((R, D), lambda i: (i, 0)),
        ],
        out_specs=[
            pl.BlockSpec((R, 1), lambda i: (i, 0)),
            pl.BlockSpec((1, D), lambda i: (0, 0)),
        ],
        out_shape=[
            jax.ShapeDtypeStruct((C, 1), jnp.float32),
            jax.ShapeDtypeStruct((1, D), jnp.float32),
        ],
        compiler_params=pltpu.CompilerParams(
            dimension_semantics=("arbitrary",),
        ),
    )(a1, a2)

    out = pl.pallas_call(
        _select_body,
        out_shape=jax.ShapeDtypeStruct((1, SIZE), jnp.float32),
    )(imp1, cs2)
    return out


def kernel(w1, w2, size):
    a1 = w1.reshape(C, D)
    a2 = w2.reshape(C, D)
    out = _run(a1, a2)
    return out.reshape(SIZE) + size * 0


# speedup vs baseline: 9.4346x; 8.6356x over previous
import jax
import jax.numpy as jnp
from jax.experimental import pallas as pl
from jax.experimental.pallas import tpu as pltpu

C = 1024
K = 9
D = C * K
R = 512
STEPS = D // R
SIZE = 512

_HI = jax.lax.Precision.HIGHEST


def _reduce_body(v1_ref, v2_ref, rs1_ref, cs2_ref):
    i = pl.program_id(0)
    rs1_ref[...] = jnp.sum(jnp.abs(v1_ref[...]), axis=1, keepdims=True)
    colpart = jnp.sum(jnp.abs(v2_ref[...]), axis=0, keepdims=True)

    @pl.when(i == 0)
    def _():
        cs2_ref[...] = colpart

    @pl.when(i > 0)
    def _():
        cs2_ref[...] = cs2_ref[...] + colpart


def _select_body(rs1_ref, cs2_ref, out_ref):
    imp1_col = rs1_ref[pl.ds(0, C), :]
    for t in range(1, K):
        imp1_col = imp1_col + rs1_ref[pl.ds(t * C, C), :]

    eye = (jax.lax.broadcasted_iota(jnp.int32, (C, C), 0)
           == jax.lax.broadcasted_iota(jnp.int32, (C, C), 1)
           ).astype(jnp.float32)
    imp2_col = jax.lax.dot_general(
        eye, cs2_ref[...], (((1,), (1,)), ((), ())), precision=_HI)
    imp_col = imp1_col + imp2_col
    imp_row = jax.lax.dot_general(
        imp_col, eye, (((0,), (0,)), ((), ())), precision=_HI)

    src_i = jax.lax.broadcasted_iota(jnp.int32, (C, C), 1)
    tgt_i = jax.lax.broadcasted_iota(jnp.int32, (C, C), 0)
    cmp = jnp.where(
        (imp_row < imp_col) | ((imp_row == imp_col) & (src_i < tgt_i)),
        1.0, 0.0)
    rank = jnp.sum(cmp, axis=1, keepdims=True)
    maskf = jnp.where(rank < float(SIZE), 1.0, 0.0)

    lower = jnp.where(src_i < tgt_i, 1.0, 0.0)
    pos = jax.lax.dot_general(
        lower, maskf, (((1,), (0,)), ((), ())), precision=_HI)

    slot = jax.lax.broadcasted_iota(jnp.int32, (C, SIZE), 1).astype(jnp.float32)
    w = maskf * jnp.where(pos == slot, 1.0, 0.0)

    lin = (-1.0 + jax.lax.broadcasted_iota(jnp.int32, (C, 1), 0)
           .astype(jnp.float32) * (2.0 / float(C - 1)))
    out_ref[...] = jax.lax.dot_general(
        lin, w, (((0,), (0,)), ((), ())), precision=_HI)


@jax.jit
def _run(w1, w2):
    v1 = jnp.transpose(w1, (2, 3, 0, 1)).reshape(D, C)
    v2 = jnp.transpose(w2, (2, 3, 0, 1)).reshape(D, C)

    rs1, cs2 = pl.pallas_call(
        _reduce_body,
        grid=(STEPS,),
        in_specs=[
            pl.BlockSpec((R, C), lambda i: (i, 0)),
            pl.BlockSpec((R, C), lambda i: (i, 0)),
        ],
        out_specs=[
            pl.BlockSpec((R, 1), lambda i: (i, 0)),
            pl.BlockSpec((1, C), lambda i: (0, 0)),
        ],
        out_shape=[
            jax.ShapeDtypeStruct((D, 1), jnp.float32),
            jax.ShapeDtypeStruct((1, C), jnp.float32),
        ],
        compiler_params=pltpu.CompilerParams(
            dimension_semantics=("arbitrary",),
        ),
    )(v1, v2)

    out = pl.pallas_call(
        _select_body,
        out_shape=jax.ShapeDtypeStruct((1, SIZE), jnp.float32),
    )(rs1, cs2)
    return out


def kernel(w1, w2, size):
    return _run(w1, w2).reshape(SIZE) + size * 0


# speedup vs baseline: 11.7027x; 1.2404x over previous
import jax
import jax.numpy as jnp
from jax.experimental import pallas as pl
from jax.experimental.pallas import tpu as pltpu

C = 1024
K = 9
D = C * K
R = 512
STEPS = D // R
SIZE = 512

_HI = jax.lax.Precision.HIGHEST


def _reduce_body(v1_ref, v2_ref, imp1_ref, cs2_ref, acc_ref):
    i = pl.program_id(0)
    half = (i % 2) * R

    rows = jnp.sum(jnp.abs(v1_ref[...]), axis=1, keepdims=True)

    @pl.when(i < 2)
    def _():
        acc_ref[pl.ds(half, R), :] = rows

    @pl.when(i >= 2)
    def _():
        acc_ref[pl.ds(half, R), :] = acc_ref[pl.ds(half, R), :] + rows

    colpart = jnp.sum(jnp.abs(v2_ref[...]), axis=0, keepdims=True)

    @pl.when(i == 0)
    def _():
        cs2_ref[...] = colpart

    @pl.when(i > 0)
    def _():
        cs2_ref[...] = cs2_ref[...] + colpart

    @pl.when(i == STEPS - 1)
    def _():
        imp1_ref[...] = acc_ref[...]


def _select_body(imp1_ref, cs2_ref, out_ref):
    imp2_row = cs2_ref[...]
    imp1_col = imp1_ref[...]
    eye = (jax.lax.broadcasted_iota(jnp.int32, (C, C), 0)
           == jax.lax.broadcasted_iota(jnp.int32, (C, C), 1)
           ).astype(jnp.float32)
    imp1_row = jax.lax.dot_general(
        imp1_col, eye, (((0,), (0,)), ((), ())), precision=_HI)
    imp2_col = jax.lax.dot_general(
        eye, imp2_row, (((1,), (1,)), ((), ())), precision=_HI)
    imp_col = imp1_col + imp2_col
    imp_row = imp1_row + imp2_row

    src_i = jax.lax.broadcasted_iota(jnp.int32, (C, C), 1)
    tgt_i = jax.lax.broadcasted_iota(jnp.int32, (C, C), 0)
    sel = (imp_row < imp_col) | ((imp_row == imp_col) & (src_i < tgt_i))
    rank = jnp.sum(jnp.where(sel, 1.0, 0.0), axis=1, keepdims=True)
    maskf = jnp.where(rank < float(SIZE), 1.0, 0.0)

    lower = jnp.where(src_i < tgt_i, 1.0, 0.0).astype(jnp.bfloat16)
    pos = jax.lax.dot_general(
        lower, maskf.astype(jnp.bfloat16), (((1,), (0,)), ((), ())),
        preferred_element_type=jnp.float32)

    slot = jax.lax.broadcasted_iota(jnp.int32, (C, SIZE), 1).astype(jnp.float32)
    w = maskf * jnp.where(pos == slot, 1.0, 0.0)
    lin = (-1.0 + jax.lax.broadcasted_iota(jnp.int32, (C, 1), 0)
           .astype(jnp.float32) * (2.0 / float(C - 1)))
    out_ref[...] = jnp.sum(w * lin, axis=0, keepdims=True)


@jax.jit
def _run(w1, w2):
    v1 = jnp.transpose(w1, (2, 3, 0, 1)).reshape(D, C)
    v2 = jnp.transpose(w2, (2, 3, 0, 1)).reshape(D, C)

    imp1, cs2 = pl.pallas_call(
        _reduce_body,
        grid=(STEPS,),
        in_specs=[
            pl.BlockSpec((R, C), lambda i: (i, 0)),
            pl.BlockSpec((R, C), lambda i: (i, 0)),
        ],
        out_specs=[
            pl.BlockSpec((C, 1), lambda i: (0, 0)),
            pl.BlockSpec((1, C), lambda i: (0, 0)),
        ],
        out_shape=[
            jax.ShapeDtypeStruct((C, 1), jnp.float32),
            jax.ShapeDtypeStruct((1, C), jnp.float32),
        ],
        scratch_shapes=[pltpu.VMEM((C, 1), jnp.float32)],
        compiler_params=pltpu.CompilerParams(
            dimension_semantics=("arbitrary",),
        ),
    )(v1, v2)

    out = pl.pallas_call(
        _select_body,
        out_shape=jax.ShapeDtypeStruct((1, SIZE), jnp.float32),
    )(imp1, cs2)
    return out


def kernel(w1, w2, size):
    return _run(w1, w2).reshape(SIZE) + size * 0
